# Initial kernel scaffold; baseline (speedup 1.0000x reference)
import jax
import jax.numpy as jnp
from jax.experimental import pallas as pl


def kernel(bboxes, scores, boxes_cache, ordered_id):
    raise NotImplementedError("write your pallas kernel here")


# speedup vs baseline: 56.5493x; 56.5493x over previous
import jax
import jax.numpy as jnp
from jax.experimental import pallas as pl
from jax.experimental.pallas import tpu as pltpu

_NUM_PROPOSALS = 300
_SCORE_THR = 0.85
_NMS_THR = 0.1
_NEG = float("-inf")

_C_PAD = 1024
_B_PAD = 20480
_TOT = _C_PAD + _B_PAD
_ROWS = _TOT // 128
_CROW_W = _NUM_PROPOSALS * 5
_BLK_ROWS = 400


def _row_gather_body(oid_ref, cache_ref, out_ref):
    del oid_ref
    out_ref[...] = cache_ref[...]


def _nms_body(x_ref, out_ref, r0, r1, r2, r3, r4):
    x1 = x_ref[0]
    y1 = x_ref[1]
    x2 = x_ref[2]
    y2 = x_ref[3]
    s = x_ref[4]
    areas = (x2 - x1) * (y2 - y1)
    rows = jax.lax.broadcasted_iota(jnp.int32, (_ROWS, 128), 0)
    lanes = jax.lax.broadcasted_iota(jnp.int32, (_ROWS, 128), 1)
    flat = rows * 128 + lanes
    big = jnp.int32(2**30)

    for r in (r0, r1, r2, r3, r4):
        r[...] = jnp.zeros_like(r)

    act = jnp.where(s > _SCORE_THR, s, _NEG)
    have = jnp.max(act) > _NEG
    gmax = jnp.max(s)
    fb = jnp.min(jnp.where(s == gmax, flat, big))
    act = jnp.where(have, act, jnp.where(flat == fb, s, _NEG))

    def cond(carry):
        _, m, _, _ = carry
        return m > _NEG

    def body(carry):
        act, m, keep, cnt = carry
        pick = jnp.min(jnp.where(act == m, flat, big))
        onehot = flat == pick
        px1 = jnp.max(jnp.where(onehot, x1, _NEG))
        py1 = jnp.max(jnp.where(onehot, y1, _NEG))
        px2 = jnp.max(jnp.where(onehot, x2, _NEG))
        py2 = jnp.max(jnp.where(onehot, y2, _NEG))
        pa = jnp.max(jnp.where(onehot, areas, _NEG))
        xx1 = jnp.maximum(px1, x1)
        yy1 = jnp.maximum(py1, y1)
        xx2 = jnp.minimum(px2, x2)
        yy2 = jnp.minimum(py2, y2)
        inter = jnp.maximum(xx2 - xx1, 0.0) * jnp.maximum(yy2 - yy1, 0.0)
        iou = inter / (pa + areas - inter + 1e-12)
        nact = jnp.where((iou > _NMS_THR) | onehot, _NEG, act)
        nkeep = jnp.where(onehot, 1.0, keep)

        @pl.when(cnt < _NUM_PROPOSALS)
        def _():
            for r, v in ((r0, px1), (r1, py1), (r2, px2), (r3, py2), (r4, m)):
                r[pl.ds(cnt, 1), :] = jnp.full((1, 1), v, jnp.float32)

        return nact, jnp.max(nact), nkeep, cnt + jnp.int32(1)

    init = (act, jnp.max(act), jnp.zeros((_ROWS, 128), jnp.float32), jnp.int32(0))
    _, _, keepf, _ = jax.lax.while_loop(cond, body, init)
    keep = keepf > 0.0
    out_ref[0] = jnp.where(keep, x1, 0.0)
    out_ref[1] = jnp.where(keep, y1, 0.0)
    out_ref[2] = jnp.where(keep, x2, 0.0)
    out_ref[3] = jnp.where(keep, y2, 0.0)
    out_ref[4] = jnp.where(keep, s, 0.0)


def _cache_copy_body(oid_ref, cache_ref, nrow_ref, out_ref):
    local = oid_ref[0] - pl.program_id(0) * _BLK_ROWS
    ridx = jax.lax.broadcasted_iota(jnp.int32, (_BLK_ROWS, 1), 0)
    out_ref[...] = jnp.where(ridx == local, nrow_ref[...], cache_ref[...])


def _plane(cvals, bvals, fill):
    return jnp.concatenate([
        cvals,
        jnp.full((_C_PAD - _NUM_PROPOSALS,), fill, jnp.float32),
        bvals,
        jnp.full((_B_PAD - bvals.shape[0],), fill, jnp.float32),
    ])


def kernel(bboxes, scores, boxes_cache, ordered_id):
    n_img = boxes_cache.shape[0]
    n_box = bboxes.shape[0]
    oid = jnp.asarray(ordered_id, jnp.int32).reshape((1,))
    cache2d = boxes_cache.reshape(n_img, _CROW_W)
    cache3d = cache2d.reshape(n_img, 1, _CROW_W)

    row = pl.pallas_call(
        _row_gather_body,
        grid_spec=pltpu.PrefetchScalarGridSpec(
            num_scalar_prefetch=1,
            grid=(1,),
            in_specs=[pl.BlockSpec((1, 1, _CROW_W), lambda i, o: (o[0], 0, 0))],
            out_specs=pl.BlockSpec((1, 1, _CROW_W), lambda i, o: (0, 0, 0)),
        ),
        out_shape=jax.ShapeDtypeStruct((1, 1, _CROW_W), jnp.float32),
    )(oid, cache3d)
    row2 = row.reshape(_NUM_PROPOSALS, 5)

    x = jnp.stack([
        _plane(row2[:, 0], bboxes[:, 0], 0.0),
        _plane(row2[:, 1], bboxes[:, 1], 0.0),
        _plane(row2[:, 2], bboxes[:, 2], 0.0),
        _plane(row2[:, 3], bboxes[:, 3], 0.0),
        _plane(row2[:, 4], scores, _NEG),
    ]).reshape(5, _ROWS, 128)

    nr_shape = jax.ShapeDtypeStruct((_NUM_PROPOSALS, 1), jnp.float32)
    outm, r0, r1, r2, r3, r4 = pl.pallas_call(
        _nms_body,
        out_shape=(
            jax.ShapeDtypeStruct((5, _ROWS, 128), jnp.float32),
            nr_shape, nr_shape, nr_shape, nr_shape, nr_shape,
        ),
    )(x)

    flatm = outm.reshape(5, _TOT)
    merged = jnp.concatenate(
        [flatm[:, :_NUM_PROPOSALS], flatm[:, _C_PAD:_C_PAD + n_box]], axis=1)
    out_boxes = merged[:4].T
    out_scores = merged[4]
    new_row = jnp.concatenate([r0, r1, r2, r3, r4], axis=1)

    new_cache2d = pl.pallas_call(
        _cache_copy_body,
        grid_spec=pltpu.PrefetchScalarGridSpec(
            num_scalar_prefetch=1,
            grid=(n_img // _BLK_ROWS,),
            in_specs=[
                pl.BlockSpec((_BLK_ROWS, _CROW_W), lambda g, o: (g, 0)),
                pl.BlockSpec((1, _CROW_W), lambda g, o: (0, 0)),
            ],
            out_specs=pl.BlockSpec((_BLK_ROWS, _CROW_W), lambda g, o: (g, 0)),
        ),
        out_shape=jax.ShapeDtypeStruct((n_img, _CROW_W), jnp.float32),
    )(oid, cache2d, new_row.reshape(1, _CROW_W))
    new_cache = new_cache2d.reshape(boxes_cache.shape)

    return out_boxes, out_scores, new_cache
